# baseline (speedup 1.0000x reference)
import functools

import jax
import jax.numpy as jnp
from jax import lax
from jax.experimental import pallas as pl
from jax.experimental.pallas import tpu as pltpu
from jax.experimental.pallas import tpu_sc as plsc

N = 10000
E = 320000
G = 100
BLK = 128
NP = 10112
GRID = NP // BLK
EP = 327680
EC = EP // 128
NSC = 2
NTILE = 16
CPT = EC // (NSC * NTILE)
K = 128
SEG = 8
STRIPE = NP // NTILE


def _leaky(v):
    return jnp.where(v > 0, v, 0.01 * v)


def _pre_body(x_ref, w1_ref, b1_ref, w2_ref, b2_ref, d_ref, wg_ref,
              h_ref, p_ref):
    h = jnp.dot(x_ref[...], w1_ref[...], preferred_element_type=jnp.float32, precision=lax.Precision.HIGHEST)
    h = _leaky(h + b1_ref[...])
    h = jnp.dot(h, w2_ref[...], preferred_element_type=jnp.float32, precision=lax.Precision.HIGHEST)
    h = _leaky(h + b2_ref[...])
    h_ref[...] = h
    dis = lax.rsqrt(d_ref[:, 0:1] + d_ref[:, 1:2])
    p_ref[...] = dis * jnp.dot(h, wg_ref[...],
                               preferred_element_type=jnp.float32, precision=lax.Precision.HIGHEST)


_pre_call = pl.pallas_call(
    _pre_body,
    grid=(GRID,),
    in_specs=[
        pl.BlockSpec((BLK, 128), lambda i: (i, 0)),
        pl.BlockSpec((128, 256), lambda i: (0, 0)),
        pl.BlockSpec((1, 256), lambda i: (0, 0)),
        pl.BlockSpec((256, 128), lambda i: (0, 0)),
        pl.BlockSpec((1, 128), lambda i: (0, 0)),
        pl.BlockSpec((BLK, 2), lambda i: (i, 0)),
        pl.BlockSpec((128, 128), lambda i: (0, 0)),
    ],
    out_specs=[
        pl.BlockSpec((BLK, 128), lambda i: (i, 0)),
        pl.BlockSpec((BLK, 128), lambda i: (i, 0)),
    ],
    out_shape=[
        jax.ShapeDtypeStruct((N, 128), jnp.float32),
        jax.ShapeDtypeStruct((N, 128), jnp.float32),
    ],
)


def _mk_body(zp_ref, p_ref, d_ref, h_ref, b_ref, w_ref, ho_ref, po_ref):
    dis = lax.rsqrt(d_ref[:, 0:1] + d_ref[:, 1:2])
    z = zp_ref[0] + zp_ref[1]
    y = dis * (z + p_ref[...]) + b_ref[...]
    hn = _leaky(y) + h_ref[...]
    ho_ref[...] = hn
    po_ref[...] = dis * jnp.dot(hn, w_ref[...], preferred_element_type=jnp.float32, precision=lax.Precision.HIGHEST)


_mk_call = pl.pallas_call(
    _mk_body,
    grid=(GRID,),
    in_specs=[
        pl.BlockSpec((NSC, BLK, 128), lambda i: (0, i, 0)),
        pl.BlockSpec((BLK, 128), lambda i: (i, 0)),
        pl.BlockSpec((BLK, 2), lambda i: (i, 0)),
        pl.BlockSpec((BLK, 128), lambda i: (i, 0)),
        pl.BlockSpec((1, 128), lambda i: (0, 0)),
        pl.BlockSpec((128, 128), lambda i: (0, 0)),
    ],
    out_specs=[
        pl.BlockSpec((BLK, 128), lambda i: (i, 0)),
        pl.BlockSpec((BLK, 128), lambda i: (i, 0)),
    ],
    out_shape=[
        jax.ShapeDtypeStruct((N, 128), jnp.float32),
        jax.ShapeDtypeStruct((N, 128), jnp.float32),
    ],
)


def _final_body(zp_ref, p_ref, d_ref, h_ref, b_ref, batch_ref, wpost_ref,
                pred_ref, sums, cnts, mx):
    i = pl.program_id(0)

    @pl.when(i == 0)
    def _():
        sums[...] = jnp.zeros_like(sums)
        cnts[...] = jnp.zeros_like(cnts)
        mx[...] = jnp.full_like(mx, -jnp.inf)

    dis = lax.rsqrt(d_ref[:, 0:1] + d_ref[:, 1:2])
    z = zp_ref[0] + zp_ref[1]
    y = dis * (z + p_ref[...]) + b_ref[...]
    h3 = _leaky(y) + h_ref[...]

    b_ids = batch_ref[...]
    valid = b_ids < G
    h3 = jnp.where(valid, h3, 0.0)
    oh_b = b_ids == lax.broadcasted_iota(jnp.int32, (BLK, G), 1)
    onehot = oh_b.astype(jnp.float32)
    sums[...] += lax.dot_general(onehot, h3, (((0,), (0,)), ((), ())),
                                 preferred_element_type=jnp.float32, precision=lax.Precision.HIGHEST)
    cnts[...] += lax.dot_general(onehot, jnp.ones((BLK, 1), jnp.float32),
                                 (((0,), (0,)), ((), ())),
                                 preferred_element_type=jnp.float32, precision=lax.Precision.HIGHEST)
    hrun = h3
    neg = jnp.float32(-3e38)
    for s in (1, 2, 4, 8, 16, 32, 64):
        hsh = jnp.concatenate(
            [jnp.full((s, 128), neg, jnp.float32), hrun[:BLK - s]], axis=0)
        bsh = jnp.concatenate(
            [jnp.full((s, 1), -1, jnp.int32), b_ids[:BLK - s]], axis=0)
        same = b_ids == bsh
        hrun = jnp.maximum(hrun, jnp.where(same, hsh, neg))
    bnext = jnp.concatenate(
        [b_ids[1:], jnp.full((1, 1), -2, jnp.int32)], axis=0)
    sel = jnp.where(b_ids != bnext, onehot, 0.0)
    mxb = lax.dot_general(sel, hrun, (((0,), (0,)), ((), ())),
                          preferred_element_type=jnp.float32, precision=lax.Precision.HIGHEST)
    present = lax.dot_general(sel, jnp.ones((BLK, 1), jnp.float32),
                              (((0,), (0,)), ((), ())),
                              preferred_element_type=jnp.float32, precision=lax.Precision.HIGHEST)
    mxb = jnp.where(present > 0, mxb, -jnp.inf)
    mx[...] = jnp.maximum(mx[...], mxb)

    @pl.when(i == GRID - 1)
    def _():
        mean = sums[...] / jnp.maximum(cnts[...], 1.0)
        pooled = mean + mx[...]
        pred_ref[...] = jnp.dot(pooled, wpost_ref[...],
                                preferred_element_type=jnp.float32, precision=lax.Precision.HIGHEST)


_final_call = pl.pallas_call(
    _final_body,
    grid=(GRID,),
    in_specs=[
        pl.BlockSpec((NSC, BLK, 128), lambda i: (0, i, 0)),
        pl.BlockSpec((BLK, 128), lambda i: (i, 0)),
        pl.BlockSpec((BLK, 2), lambda i: (i, 0)),
        pl.BlockSpec((BLK, 128), lambda i: (i, 0)),
        pl.BlockSpec((1, 128), lambda i: (0, 0)),
        pl.BlockSpec((BLK, 1), lambda i: (i, 0)),
        pl.BlockSpec((128, 1), lambda i: (0, 0)),
    ],
    out_specs=pl.BlockSpec((G, 1), lambda i: (0, 0)),
    out_shape=jax.ShapeDtypeStruct((G, 1), jnp.float32),
    scratch_shapes=[
        pltpu.VMEM((G, 128), jnp.float32),
        pltpu.VMEM((G, 1), jnp.float32),
        pltpu.VMEM((G, 128), jnp.float32),
    ],
)


def _sc_mesh():
    return plsc.VectorSubcoreMesh(
        core_axis_name="c", subcore_axis_name="s",
        num_cores=NSC, num_subcores=NTILE)


@functools.partial(
    pl.kernel,
    out_type=jax.ShapeDtypeStruct((NSC * NP,), jnp.float32),
    mesh=plsc.VectorSubcoreMesh(core_axis_name="c", subcore_axis_name="s",
                                num_cores=NSC, num_subcores=NTILE),
    scratch_types=[
        pltpu.VMEM((CPT, K), jnp.int32),
        pltpu.VMEM((K,), jnp.float32),
        pltpu.VMEM((STRIPE,), jnp.float32),
        pltpu.VMEM_SHARED((NP,), jnp.float32),
    ],
)
def _deg_call(dst_hbm, init_hbm, out_hbm, didx, ones_v, stage, acc):
    cid = lax.axis_index("c")
    sid = lax.axis_index("s")
    wid = cid * NTILE + sid
    for j in range(K // 16):
        ones_v[pl.ds(j * 16, 16)] = jnp.ones((16,), jnp.float32)
    pltpu.sync_copy(init_hbm.at[pl.ds(cid * STRIPE, STRIPE)], stage)
    pltpu.sync_copy(stage, acc.at[pl.ds(sid * STRIPE, STRIPE)])
    pltpu.sync_copy(dst_hbm.at[pl.ds(wid * CPT, CPT)], didx)
    plsc.subcore_barrier()

    def body(c, carry):
        pltpu.sync_copy(ones_v, acc.at[didx.at[c]], add=True)
        return carry

    lax.fori_loop(0, CPT, body, 0)
    plsc.subcore_barrier()
    pltpu.sync_copy(acc.at[pl.ds(sid * STRIPE, STRIPE)], stage)
    pltpu.sync_copy(stage, out_hbm.at[pl.ds(cid * NP + sid * STRIPE, STRIPE)])


@functools.partial(
    pl.kernel,
    out_type=jax.ShapeDtypeStruct((NSC, NP, 128), jnp.float32),
    mesh=plsc.VectorSubcoreMesh(core_axis_name="c", subcore_axis_name="s",
                                num_cores=NSC, num_subcores=NTILE),
    scratch_types=[
        pltpu.VMEM((SEG, K), jnp.int32),
        pltpu.VMEM((SEG, K), jnp.int32),
        pltpu.VMEM((SEG, K), jnp.int32),
        pltpu.VMEM((SEG, K), jnp.int32),
        pltpu.VMEM((K, 128), jnp.float32),
        pltpu.VMEM((K, 128), jnp.float32),
        pltpu.VMEM_SHARED((NP, 128), jnp.float32),
        pltpu.SemaphoreType.DMA,
        pltpu.SemaphoreType.DMA,
        pltpu.SemaphoreType.DMA,
        pltpu.SemaphoreType.DMA,
    ],
)
def _msg_call(p_hbm, src_hbm, dst_hbm, zeros_hbm, out_hbm,
              sidx_a, didx_a, sidx_b, didx_b, rows0, rows1, acc,
              sem0, sem1, sem_a, sem_b):
    cid = lax.axis_index("c")
    sid = lax.axis_index("s")
    wid = cid * NTILE + sid
    bufs = (rows0, rows1)
    sems = (sem0, sem1)
    ebase = wid * CPT
    pltpu.async_copy(src_hbm.at[pl.ds(ebase, SEG)], sidx_a, sem_a)
    pltpu.async_copy(dst_hbm.at[pl.ds(ebase, SEG)], didx_a, sem_a)
    pltpu.sync_copy(zeros_hbm, rows0)
    zd = []
    for off in (0, K, 2 * K, 3 * K, 4 * K):
        c = min(K, STRIPE - off)
        zd.append(pltpu.async_copy(rows0.at[pl.ds(0, c)],
                                   acc.at[pl.ds(sid * STRIPE + off, c)],
                                   sem1))
    for d in zd:
        d.wait()
    plsc.subcore_barrier()

    def process(sidx, didx):
        d = [None] * SEG
        d[0] = pltpu.async_copy(p_hbm.at[sidx.at[0]], rows0, sem0)
        d[1] = pltpu.async_copy(p_hbm.at[sidx.at[1]], rows1, sem1)
        for b in range(SEG):
            d[b].wait()
            pltpu.sync_copy(bufs[b % 2], acc.at[didx.at[b]], add=True)
            if b + 2 < SEG:
                d[b + 2] = pltpu.async_copy(
                    p_hbm.at[sidx.at[b + 2]], bufs[b % 2], sems[b % 2])

    npair = CPT // (2 * SEG)

    def body(j, carry):
        sa = ebase + 2 * j * SEG
        sb = sa + SEG
        pltpu.make_async_copy(
            src_hbm.at[pl.ds(sa, SEG)], sidx_a, sem_a).wait()
        pltpu.make_async_copy(
            dst_hbm.at[pl.ds(sa, SEG)], didx_a, sem_a).wait()
        pltpu.async_copy(src_hbm.at[pl.ds(sb, SEG)], sidx_b, sem_b)
        pltpu.async_copy(dst_hbm.at[pl.ds(sb, SEG)], didx_b, sem_b)
        process(sidx_a, didx_a)

        @pl.when(j + 1 < npair)
        def _():
            sn = sa + 2 * SEG
            pltpu.async_copy(src_hbm.at[pl.ds(sn, SEG)], sidx_a, sem_a)
            pltpu.async_copy(dst_hbm.at[pl.ds(sn, SEG)], didx_a, sem_a)

        pltpu.make_async_copy(
            src_hbm.at[pl.ds(sb, SEG)], sidx_b, sem_b).wait()
        pltpu.make_async_copy(
            dst_hbm.at[pl.ds(sb, SEG)], didx_b, sem_b).wait()
        process(sidx_b, didx_b)
        return carry

    lax.fori_loop(0, npair, body, 0)
    plsc.subcore_barrier()
    for off in (0, K, 2 * K, 3 * K, 4 * K):
        c = min(K, STRIPE - off)
        pltpu.sync_copy(acc.at[pl.ds(sid * STRIPE + off, c)],
                        rows0.at[pl.ds(0, c)])
        pltpu.sync_copy(rows0.at[pl.ds(0, c)],
                        out_hbm.at[cid, pl.ds(sid * STRIPE + off, c)])


def kernel(x, edge_index, batch, W_pre1, b_pre1, W_pre2, b_pre2,
           W_g0, b_g0, W_g1, b_g1, W_g2, b_g2, W_post):
    src = edge_index[0].astype(jnp.int32)
    dst = edge_index[1].astype(jnp.int32)
    padn = EP - E
    pad_src = (jnp.arange(padn, dtype=jnp.int32) * 37) % N
    pad_dst = N + jnp.arange(padn, dtype=jnp.int32) % (NP - N)
    src2 = jnp.concatenate([src, pad_src]).reshape(EC, K)
    dst2 = jnp.concatenate([dst, pad_dst]).reshape(EC, K)
    zeros_stripe = jnp.zeros((K, 128), jnp.float32)
    deg_init = jnp.concatenate([jnp.ones((STRIPE,), jnp.float32),
                                jnp.zeros((STRIPE,), jnp.float32)])
    batch_p = jnp.concatenate([
        batch.astype(jnp.int32),
        jnp.full((NP - N,), 127, jnp.int32)]).reshape(NP, 1)
    b1 = b_pre1.reshape(1, -1)
    b2 = b_pre2.reshape(1, -1)
    bg0 = b_g0.reshape(1, -1)
    bg1 = b_g1.reshape(1, -1)
    bg2 = b_g2.reshape(1, -1)

    degp = _deg_call(dst2, deg_init)
    deg_t = jnp.transpose(degp.reshape(NSC, NP))
    h0, p0 = _pre_call(x, W_pre1, b1, W_pre2, b2, deg_t, W_g0)
    zp0 = _msg_call(p0, src2, dst2, zeros_stripe)
    h1, p1 = _mk_call(zp0, p0, deg_t, h0, bg0, W_g1)
    zp1 = _msg_call(p1, src2, dst2, zeros_stripe)
    h2, p2 = _mk_call(zp1, p1, deg_t, h1, bg1, W_g2)
    zp2 = _msg_call(p2, src2, dst2, zeros_stripe)
    pred = _final_call(zp2, p2, deg_t, h2, bg2, batch_p, W_post)
    return pred.reshape(-1, 10)


# speedup vs baseline: 1.2423x; 1.2423x over previous
import functools

import jax
import jax.numpy as jnp
from jax import lax
from jax.experimental import pallas as pl
from jax.experimental.pallas import tpu as pltpu
from jax.experimental.pallas import tpu_sc as plsc

N = 10000
E = 320000
G = 100
BLK = 256
NP = 10112
GRID = 40
EP = 327680
EC = EP // 128
NSC = 2
NTILE = 16
CPT = EC // (NSC * NTILE)
K = 128
SEG = 8
STRIPE = NP // NTILE


def _leaky(v):
    return jnp.where(v > 0, v, 0.01 * v)


def _pre_body(x_ref, w1_ref, b1_ref, w2_ref, b2_ref, d_ref, wg_ref,
              h_ref, p_ref):
    h = jnp.dot(x_ref[...], w1_ref[...], preferred_element_type=jnp.float32)
    h = _leaky(h + b1_ref[...])
    h = jnp.dot(h, w2_ref[...], preferred_element_type=jnp.float32)
    h = _leaky(h + b2_ref[...])
    h_ref[...] = h
    dis = lax.rsqrt(d_ref[:, 0:1] + d_ref[:, 1:2])
    p_ref[...] = dis * jnp.dot(h, wg_ref[...],
                               preferred_element_type=jnp.float32)


_pre_call = pl.pallas_call(
    _pre_body,
    grid=(GRID,),
    in_specs=[
        pl.BlockSpec((BLK, 128), lambda i: (i, 0)),
        pl.BlockSpec((128, 256), lambda i: (0, 0)),
        pl.BlockSpec((1, 256), lambda i: (0, 0)),
        pl.BlockSpec((256, 128), lambda i: (0, 0)),
        pl.BlockSpec((1, 128), lambda i: (0, 0)),
        pl.BlockSpec((BLK, 2), lambda i: (i, 0)),
        pl.BlockSpec((128, 128), lambda i: (0, 0)),
    ],
    out_specs=[
        pl.BlockSpec((BLK, 128), lambda i: (i, 0)),
        pl.BlockSpec((BLK, 128), lambda i: (i, 0)),
    ],
    out_shape=[
        jax.ShapeDtypeStruct((N, 128), jnp.float32),
        jax.ShapeDtypeStruct((N, 128), jnp.float32),
    ],
)


def _mk_body(zp_ref, p_ref, d_ref, h_ref, b_ref, w_ref, ho_ref, po_ref):
    dis = lax.rsqrt(d_ref[:, 0:1] + d_ref[:, 1:2])
    z = zp_ref[0] + zp_ref[1]
    y = dis * (z + p_ref[...]) + b_ref[...]
    hn = _leaky(y) + h_ref[...]
    ho_ref[...] = hn
    po_ref[...] = dis * jnp.dot(hn, w_ref[...], preferred_element_type=jnp.float32)


_mk_call = pl.pallas_call(
    _mk_body,
    grid=(GRID,),
    in_specs=[
        pl.BlockSpec((NSC, BLK, 128), lambda i: (0, i, 0)),
        pl.BlockSpec((BLK, 128), lambda i: (i, 0)),
        pl.BlockSpec((BLK, 2), lambda i: (i, 0)),
        pl.BlockSpec((BLK, 128), lambda i: (i, 0)),
        pl.BlockSpec((1, 128), lambda i: (0, 0)),
        pl.BlockSpec((128, 128), lambda i: (0, 0)),
    ],
    out_specs=[
        pl.BlockSpec((BLK, 128), lambda i: (i, 0)),
        pl.BlockSpec((BLK, 128), lambda i: (i, 0)),
    ],
    out_shape=[
        jax.ShapeDtypeStruct((N, 128), jnp.float32),
        jax.ShapeDtypeStruct((N, 128), jnp.float32),
    ],
)


def _final_body(zp_ref, p_ref, d_ref, h_ref, b_ref, batch_ref, wpost_ref,
                pred_ref, sums, cnts, mx):
    i = pl.program_id(0)

    @pl.when(i == 0)
    def _():
        sums[...] = jnp.zeros_like(sums)
        cnts[...] = jnp.zeros_like(cnts)
        mx[...] = jnp.full_like(mx, -jnp.inf)

    dis = lax.rsqrt(d_ref[:, 0:1] + d_ref[:, 1:2])
    z = zp_ref[0] + zp_ref[1]
    y = dis * (z + p_ref[...]) + b_ref[...]
    h3 = _leaky(y) + h_ref[...]

    b_ids = batch_ref[...]
    valid = b_ids < G
    h3 = jnp.where(valid, h3, 0.0)
    oh_b = b_ids == lax.broadcasted_iota(jnp.int32, (BLK, G), 1)
    onehot = oh_b.astype(jnp.float32)
    sums[...] += lax.dot_general(onehot, h3, (((0,), (0,)), ((), ())),
                                 preferred_element_type=jnp.float32)
    cnts[...] += lax.dot_general(onehot, jnp.ones((BLK, 1), jnp.float32),
                                 (((0,), (0,)), ((), ())),
                                 preferred_element_type=jnp.float32)
    hrun = h3
    neg = jnp.float32(-3e38)
    for s in (1, 2, 4, 8, 16, 32, 64, 128):
        hsh = jnp.concatenate(
            [jnp.full((s, 128), neg, jnp.float32), hrun[:BLK - s]], axis=0)
        bsh = jnp.concatenate(
            [jnp.full((s, 1), -1, jnp.int32), b_ids[:BLK - s]], axis=0)
        same = b_ids == bsh
        hrun = jnp.maximum(hrun, jnp.where(same, hsh, neg))
    bnext = jnp.concatenate(
        [b_ids[1:], jnp.full((1, 1), -2, jnp.int32)], axis=0)
    sel = jnp.where(b_ids != bnext, onehot, 0.0)
    mxb = lax.dot_general(sel, hrun, (((0,), (0,)), ((), ())),
                          preferred_element_type=jnp.float32)
    present = lax.dot_general(sel, jnp.ones((BLK, 1), jnp.float32),
                              (((0,), (0,)), ((), ())),
                              preferred_element_type=jnp.float32)
    mxb = jnp.where(present > 0, mxb, -jnp.inf)
    mx[...] = jnp.maximum(mx[...], mxb)

    @pl.when(i == GRID - 1)
    def _():
        mean = sums[...] / jnp.maximum(cnts[...], 1.0)
        pooled = mean + mx[...]
        pred_ref[...] = jnp.dot(pooled, wpost_ref[...],
                                preferred_element_type=jnp.float32)


_final_call = pl.pallas_call(
    _final_body,
    grid=(GRID,),
    in_specs=[
        pl.BlockSpec((NSC, BLK, 128), lambda i: (0, i, 0)),
        pl.BlockSpec((BLK, 128), lambda i: (i, 0)),
        pl.BlockSpec((BLK, 2), lambda i: (i, 0)),
        pl.BlockSpec((BLK, 128), lambda i: (i, 0)),
        pl.BlockSpec((1, 128), lambda i: (0, 0)),
        pl.BlockSpec((BLK, 1), lambda i: (i, 0)),
        pl.BlockSpec((128, 1), lambda i: (0, 0)),
    ],
    out_specs=pl.BlockSpec((G, 1), lambda i: (0, 0)),
    out_shape=jax.ShapeDtypeStruct((G, 1), jnp.float32),
    scratch_shapes=[
        pltpu.VMEM((G, 128), jnp.float32),
        pltpu.VMEM((G, 1), jnp.float32),
        pltpu.VMEM((G, 128), jnp.float32),
    ],
)


def _sc_mesh():
    return plsc.VectorSubcoreMesh(
        core_axis_name="c", subcore_axis_name="s",
        num_cores=NSC, num_subcores=NTILE)


@functools.partial(
    pl.kernel,
    out_type=jax.ShapeDtypeStruct((NSC * NP,), jnp.float32),
    mesh=plsc.VectorSubcoreMesh(core_axis_name="c", subcore_axis_name="s",
                                num_cores=NSC, num_subcores=NTILE),
    scratch_types=[
        pltpu.VMEM((CPT, K), jnp.int32),
        pltpu.VMEM((K,), jnp.float32),
        pltpu.VMEM((STRIPE,), jnp.float32),
        pltpu.VMEM_SHARED((NP,), jnp.float32),
    ],
)
def _deg_call(dst_hbm, init_hbm, out_hbm, didx, ones_v, stage, acc):
    cid = lax.axis_index("c")
    sid = lax.axis_index("s")
    wid = cid * NTILE + sid
    for j in range(K // 16):
        ones_v[pl.ds(j * 16, 16)] = jnp.ones((16,), jnp.float32)
    pltpu.sync_copy(init_hbm.at[pl.ds(cid * STRIPE, STRIPE)], stage)
    pltpu.sync_copy(stage, acc.at[pl.ds(sid * STRIPE, STRIPE)])
    pltpu.sync_copy(dst_hbm.at[pl.ds(wid * CPT, CPT)], didx)
    plsc.subcore_barrier()

    def body(c, carry):
        pltpu.sync_copy(ones_v, acc.at[didx.at[c]], add=True)
        return carry

    lax.fori_loop(0, CPT, body, 0)
    plsc.subcore_barrier()
    pltpu.sync_copy(acc.at[pl.ds(sid * STRIPE, STRIPE)], stage)
    pltpu.sync_copy(stage, out_hbm.at[pl.ds(cid * NP + sid * STRIPE, STRIPE)])


@functools.partial(
    pl.kernel,
    out_type=jax.ShapeDtypeStruct((NSC, NP, 128), jnp.float32),
    mesh=plsc.VectorSubcoreMesh(core_axis_name="c", subcore_axis_name="s",
                                num_cores=NSC, num_subcores=NTILE),
    scratch_types=[
        pltpu.VMEM((SEG, K), jnp.int32),
        pltpu.VMEM((SEG, K), jnp.int32),
        pltpu.VMEM((SEG, K), jnp.int32),
        pltpu.VMEM((SEG, K), jnp.int32),
        pltpu.VMEM((K, 128), jnp.float32),
        pltpu.VMEM((K, 128), jnp.float32),
        pltpu.VMEM_SHARED((NP, 128), jnp.float32),
        pltpu.SemaphoreType.DMA,
        pltpu.SemaphoreType.DMA,
        pltpu.SemaphoreType.DMA,
        pltpu.SemaphoreType.DMA,
    ],
)
def _msg_call(p_hbm, src_hbm, dst_hbm, zeros_hbm, out_hbm,
              sidx_a, didx_a, sidx_b, didx_b, rows0, rows1, acc,
              sem0, sem1, sem_a, sem_b):
    cid = lax.axis_index("c")
    sid = lax.axis_index("s")
    wid = cid * NTILE + sid
    bufs = (rows0, rows1)
    sems = (sem0, sem1)
    ebase = wid * CPT
    pltpu.async_copy(src_hbm.at[pl.ds(ebase, SEG)], sidx_a, sem_a)
    pltpu.async_copy(dst_hbm.at[pl.ds(ebase, SEG)], didx_a, sem_a)
    pltpu.sync_copy(zeros_hbm, rows0)
    zd = []
    for off in (0, K, 2 * K, 3 * K, 4 * K):
        c = min(K, STRIPE - off)
        zd.append(pltpu.async_copy(rows0.at[pl.ds(0, c)],
                                   acc.at[pl.ds(sid * STRIPE + off, c)],
                                   sem1))
    for d in zd:
        d.wait()
    plsc.subcore_barrier()

    def process(sidx, didx):
        d = [None] * SEG
        d[0] = pltpu.async_copy(p_hbm.at[sidx.at[0]], rows0, sem0)
        d[1] = pltpu.async_copy(p_hbm.at[sidx.at[1]], rows1, sem1)
        for b in range(SEG):
            d[b].wait()
            pltpu.sync_copy(bufs[b % 2], acc.at[didx.at[b]], add=True)
            if b + 2 < SEG:
                d[b + 2] = pltpu.async_copy(
                    p_hbm.at[sidx.at[b + 2]], bufs[b % 2], sems[b % 2])

    npair = CPT // (2 * SEG)

    def body(j, carry):
        sa = ebase + 2 * j * SEG
        sb = sa + SEG
        pltpu.make_async_copy(
            src_hbm.at[pl.ds(sa, SEG)], sidx_a, sem_a).wait()
        pltpu.make_async_copy(
            dst_hbm.at[pl.ds(sa, SEG)], didx_a, sem_a).wait()
        pltpu.async_copy(src_hbm.at[pl.ds(sb, SEG)], sidx_b, sem_b)
        pltpu.async_copy(dst_hbm.at[pl.ds(sb, SEG)], didx_b, sem_b)
        process(sidx_a, didx_a)

        @pl.when(j + 1 < npair)
        def _():
            sn = sa + 2 * SEG
            pltpu.async_copy(src_hbm.at[pl.ds(sn, SEG)], sidx_a, sem_a)
            pltpu.async_copy(dst_hbm.at[pl.ds(sn, SEG)], didx_a, sem_a)

        pltpu.make_async_copy(
            src_hbm.at[pl.ds(sb, SEG)], sidx_b, sem_b).wait()
        pltpu.make_async_copy(
            dst_hbm.at[pl.ds(sb, SEG)], didx_b, sem_b).wait()
        process(sidx_b, didx_b)
        return carry

    lax.fori_loop(0, npair, body, 0)
    plsc.subcore_barrier()
    for off in (0, K, 2 * K, 3 * K, 4 * K):
        c = min(K, STRIPE - off)
        pltpu.sync_copy(acc.at[pl.ds(sid * STRIPE + off, c)],
                        rows0.at[pl.ds(0, c)])
        pltpu.sync_copy(rows0.at[pl.ds(0, c)],
                        out_hbm.at[cid, pl.ds(sid * STRIPE + off, c)])


def kernel(x, edge_index, batch, W_pre1, b_pre1, W_pre2, b_pre2,
           W_g0, b_g0, W_g1, b_g1, W_g2, b_g2, W_post):
    src = edge_index[0].astype(jnp.int32)
    dst = edge_index[1].astype(jnp.int32)
    padn = EP - E
    pad_src = (jnp.arange(padn, dtype=jnp.int32) * 37) % N
    pad_dst = N + jnp.arange(padn, dtype=jnp.int32) % (NP - N)
    src2 = jnp.concatenate([src, pad_src]).reshape(EC, K)
    dst2 = jnp.concatenate([dst, pad_dst]).reshape(EC, K)
    zeros_stripe = jnp.zeros((K, 128), jnp.float32)
    deg_init = jnp.concatenate([jnp.ones((STRIPE,), jnp.float32),
                                jnp.zeros((STRIPE,), jnp.float32)])
    batch_p = jnp.concatenate([
        batch.astype(jnp.int32),
        jnp.full((GRID * BLK - N,), 127, jnp.int32)]).reshape(GRID * BLK, 1)
    b1 = b_pre1.reshape(1, -1)
    b2 = b_pre2.reshape(1, -1)
    bg0 = b_g0.reshape(1, -1)
    bg1 = b_g1.reshape(1, -1)
    bg2 = b_g2.reshape(1, -1)

    degp = _deg_call(dst2, deg_init)
    deg_t = jnp.transpose(degp.reshape(NSC, NP))
    h0, p0 = _pre_call(x, W_pre1, b1, W_pre2, b2, deg_t, W_g0)
    zp0 = _msg_call(p0, src2, dst2, zeros_stripe)
    h1, p1 = _mk_call(zp0, p0, deg_t, h0, bg0, W_g1)
    zp1 = _msg_call(p1, src2, dst2, zeros_stripe)
    h2, p2 = _mk_call(zp1, p1, deg_t, h1, bg1, W_g2)
    zp2 = _msg_call(p2, src2, dst2, zeros_stripe)
    pred = _final_call(zp2, p2, deg_t, h2, bg2, batch_p, W_post)
    return pred.reshape(-1, 10)


# speedup vs baseline: 1.3444x; 1.0822x over previous
import functools

import jax
import jax.numpy as jnp
from jax import lax
from jax.experimental import pallas as pl
from jax.experimental.pallas import tpu as pltpu
from jax.experimental.pallas import tpu_sc as plsc

N = 10000
E = 320000
G = 100
BLK = 512
NP = 10112
GRID = 20
EP = 327680
EC = EP // 128
NSC = 2
NTILE = 16
CPT = EC // (NSC * NTILE)
K = 128
SEG = 8
STRIPE = NP // NTILE


def _leaky(v):
    return jnp.where(v > 0, v, 0.01 * v)


def _pre_body(x_ref, w1_ref, b1_ref, w2_ref, b2_ref, d_ref, wg_ref,
              h_ref, p_ref):
    h = jnp.dot(x_ref[...], w1_ref[...], preferred_element_type=jnp.float32)
    h = _leaky(h + b1_ref[...])
    h = jnp.dot(h, w2_ref[...], preferred_element_type=jnp.float32)
    h = _leaky(h + b2_ref[...])
    h_ref[...] = h
    dis = lax.rsqrt(d_ref[:, 0:1] + d_ref[:, 1:2])
    p_ref[...] = dis * jnp.dot(h, wg_ref[...],
                               preferred_element_type=jnp.float32)


_pre_call = pl.pallas_call(
    _pre_body,
    grid=(GRID,),
    in_specs=[
        pl.BlockSpec((BLK, 128), lambda i: (i, 0)),
        pl.BlockSpec((128, 256), lambda i: (0, 0)),
        pl.BlockSpec((1, 256), lambda i: (0, 0)),
        pl.BlockSpec((256, 128), lambda i: (0, 0)),
        pl.BlockSpec((1, 128), lambda i: (0, 0)),
        pl.BlockSpec((BLK, 2), lambda i: (i, 0)),
        pl.BlockSpec((128, 128), lambda i: (0, 0)),
    ],
    out_specs=[
        pl.BlockSpec((BLK, 128), lambda i: (i, 0)),
        pl.BlockSpec((BLK, 128), lambda i: (i, 0)),
    ],
    out_shape=[
        jax.ShapeDtypeStruct((N, 128), jnp.float32),
        jax.ShapeDtypeStruct((N, 128), jnp.float32),
    ],
)


def _mk_body(zp_ref, p_ref, d_ref, h_ref, b_ref, w_ref, ho_ref, po_ref):
    dis = lax.rsqrt(d_ref[:, 0:1] + d_ref[:, 1:2])
    z = zp_ref[0] + zp_ref[1]
    y = dis * (z + p_ref[...]) + b_ref[...]
    hn = _leaky(y) + h_ref[...]
    ho_ref[...] = hn
    po_ref[...] = dis * jnp.dot(hn, w_ref[...], preferred_element_type=jnp.float32)


_mk_call = pl.pallas_call(
    _mk_body,
    grid=(GRID,),
    in_specs=[
        pl.BlockSpec((NSC, BLK, 128), lambda i: (0, i, 0)),
        pl.BlockSpec((BLK, 128), lambda i: (i, 0)),
        pl.BlockSpec((BLK, 2), lambda i: (i, 0)),
        pl.BlockSpec((BLK, 128), lambda i: (i, 0)),
        pl.BlockSpec((1, 128), lambda i: (0, 0)),
        pl.BlockSpec((128, 128), lambda i: (0, 0)),
    ],
    out_specs=[
        pl.BlockSpec((BLK, 128), lambda i: (i, 0)),
        pl.BlockSpec((BLK, 128), lambda i: (i, 0)),
    ],
    out_shape=[
        jax.ShapeDtypeStruct((N, 128), jnp.float32),
        jax.ShapeDtypeStruct((N, 128), jnp.float32),
    ],
)


def _final_body(zp_ref, p_ref, d_ref, h_ref, b_ref, batch_ref, wpost_ref,
                pred_ref, sums, cnts, mx):
    i = pl.program_id(0)

    @pl.when(i == 0)
    def _():
        sums[...] = jnp.zeros_like(sums)
        cnts[...] = jnp.zeros_like(cnts)
        mx[...] = jnp.full_like(mx, -jnp.inf)

    dis = lax.rsqrt(d_ref[:, 0:1] + d_ref[:, 1:2])
    z = zp_ref[0] + zp_ref[1]
    y = dis * (z + p_ref[...]) + b_ref[...]
    h3 = _leaky(y) + h_ref[...]

    b_ids = batch_ref[...]
    valid = b_ids < G
    h3 = jnp.where(valid, h3, 0.0)
    oh_b = b_ids == lax.broadcasted_iota(jnp.int32, (BLK, G), 1)
    onehot = oh_b.astype(jnp.float32)
    sums[...] += lax.dot_general(onehot, h3, (((0,), (0,)), ((), ())),
                                 preferred_element_type=jnp.float32)
    cnts[...] += lax.dot_general(onehot, jnp.ones((BLK, 1), jnp.float32),
                                 (((0,), (0,)), ((), ())),
                                 preferred_element_type=jnp.float32)
    hrun = h3
    neg = jnp.float32(-3e38)
    for s in (1, 2, 4, 8, 16, 32, 64, 128, 256):
        hsh = jnp.concatenate(
            [jnp.full((s, 128), neg, jnp.float32), hrun[:BLK - s]], axis=0)
        bsh = jnp.concatenate(
            [jnp.full((s, 1), -1, jnp.int32), b_ids[:BLK - s]], axis=0)
        same = b_ids == bsh
        hrun = jnp.maximum(hrun, jnp.where(same, hsh, neg))
    bnext = jnp.concatenate(
        [b_ids[1:], jnp.full((1, 1), -2, jnp.int32)], axis=0)
    sel = jnp.where(b_ids != bnext, onehot, 0.0)
    mxb = lax.dot_general(sel, hrun, (((0,), (0,)), ((), ())),
                          preferred_element_type=jnp.float32)
    present = lax.dot_general(sel, jnp.ones((BLK, 1), jnp.float32),
                              (((0,), (0,)), ((), ())),
                              preferred_element_type=jnp.float32)
    mxb = jnp.where(present > 0, mxb, -jnp.inf)
    mx[...] = jnp.maximum(mx[...], mxb)

    @pl.when(i == GRID - 1)
    def _():
        mean = sums[...] / jnp.maximum(cnts[...], 1.0)
        pooled = mean + mx[...]
        pred_ref[...] = jnp.dot(pooled, wpost_ref[...],
                                preferred_element_type=jnp.float32)


_final_call = pl.pallas_call(
    _final_body,
    grid=(GRID,),
    in_specs=[
        pl.BlockSpec((NSC, BLK, 128), lambda i: (0, i, 0)),
        pl.BlockSpec((BLK, 128), lambda i: (i, 0)),
        pl.BlockSpec((BLK, 2), lambda i: (i, 0)),
        pl.BlockSpec((BLK, 128), lambda i: (i, 0)),
        pl.BlockSpec((1, 128), lambda i: (0, 0)),
        pl.BlockSpec((BLK, 1), lambda i: (i, 0)),
        pl.BlockSpec((128, 1), lambda i: (0, 0)),
    ],
    out_specs=pl.BlockSpec((G, 1), lambda i: (0, 0)),
    out_shape=jax.ShapeDtypeStruct((G, 1), jnp.float32),
    scratch_shapes=[
        pltpu.VMEM((G, 128), jnp.float32),
        pltpu.VMEM((G, 1), jnp.float32),
        pltpu.VMEM((G, 128), jnp.float32),
    ],
)


def _sc_mesh():
    return plsc.VectorSubcoreMesh(
        core_axis_name="c", subcore_axis_name="s",
        num_cores=NSC, num_subcores=NTILE)


@functools.partial(
    pl.kernel,
    out_type=jax.ShapeDtypeStruct((NSC * NP,), jnp.float32),
    mesh=plsc.VectorSubcoreMesh(core_axis_name="c", subcore_axis_name="s",
                                num_cores=NSC, num_subcores=NTILE),
    scratch_types=[
        pltpu.VMEM((CPT, K), jnp.int32),
        pltpu.VMEM((K,), jnp.float32),
        pltpu.VMEM((STRIPE,), jnp.float32),
        pltpu.VMEM_SHARED((NP,), jnp.float32),
    ],
)
def _deg_call(dst_hbm, init_hbm, out_hbm, didx, ones_v, stage, acc):
    cid = lax.axis_index("c")
    sid = lax.axis_index("s")
    wid = cid * NTILE + sid
    for j in range(K // 16):
        ones_v[pl.ds(j * 16, 16)] = jnp.ones((16,), jnp.float32)
    pltpu.sync_copy(init_hbm.at[pl.ds(cid * STRIPE, STRIPE)], stage)
    pltpu.sync_copy(stage, acc.at[pl.ds(sid * STRIPE, STRIPE)])
    pltpu.sync_copy(dst_hbm.at[pl.ds(wid * CPT, CPT)], didx)
    plsc.subcore_barrier()

    def body(c, carry):
        pltpu.sync_copy(ones_v, acc.at[didx.at[c]], add=True)
        return carry

    lax.fori_loop(0, CPT, body, 0)
    plsc.subcore_barrier()
    pltpu.sync_copy(acc.at[pl.ds(sid * STRIPE, STRIPE)], stage)
    pltpu.sync_copy(stage, out_hbm.at[pl.ds(cid * NP + sid * STRIPE, STRIPE)])


@functools.partial(
    pl.kernel,
    out_type=jax.ShapeDtypeStruct((NSC, NP, 128), jnp.float32),
    mesh=plsc.VectorSubcoreMesh(core_axis_name="c", subcore_axis_name="s",
                                num_cores=NSC, num_subcores=NTILE),
    scratch_types=[
        pltpu.VMEM((SEG, K), jnp.int32),
        pltpu.VMEM((SEG, K), jnp.int32),
        pltpu.VMEM((SEG, K), jnp.int32),
        pltpu.VMEM((SEG, K), jnp.int32),
        pltpu.VMEM((K, 128), jnp.float32),
        pltpu.VMEM((K, 128), jnp.float32),
        pltpu.VMEM_SHARED((NP, 128), jnp.float32),
        pltpu.SemaphoreType.DMA,
        pltpu.SemaphoreType.DMA,
        pltpu.SemaphoreType.DMA,
        pltpu.SemaphoreType.DMA,
    ],
)
def _msg_call(p_hbm, src_hbm, dst_hbm, zeros_hbm, out_hbm,
              sidx_a, didx_a, sidx_b, didx_b, rows0, rows1, acc,
              sem0, sem1, sem_a, sem_b):
    cid = lax.axis_index("c")
    sid = lax.axis_index("s")
    wid = cid * NTILE + sid
    bufs = (rows0, rows1)
    sems = (sem0, sem1)
    ebase = wid * CPT
    pltpu.async_copy(src_hbm.at[pl.ds(ebase, SEG)], sidx_a, sem_a)
    pltpu.async_copy(dst_hbm.at[pl.ds(ebase, SEG)], didx_a, sem_a)
    pltpu.sync_copy(zeros_hbm, rows0)
    zd = []
    for off in (0, K, 2 * K, 3 * K, 4 * K):
        c = min(K, STRIPE - off)
        zd.append(pltpu.async_copy(rows0.at[pl.ds(0, c)],
                                   acc.at[pl.ds(sid * STRIPE + off, c)],
                                   sem1))
    for d in zd:
        d.wait()
    plsc.subcore_barrier()

    def process(sidx, didx):
        d = [None] * SEG
        d[0] = pltpu.async_copy(p_hbm.at[sidx.at[0]], rows0, sem0)
        d[1] = pltpu.async_copy(p_hbm.at[sidx.at[1]], rows1, sem1)
        for b in range(SEG):
            d[b].wait()
            pltpu.sync_copy(bufs[b % 2], acc.at[didx.at[b]], add=True)
            if b + 2 < SEG:
                d[b + 2] = pltpu.async_copy(
                    p_hbm.at[sidx.at[b + 2]], bufs[b % 2], sems[b % 2])

    npair = CPT // (2 * SEG)

    def body(j, carry):
        sa = ebase + 2 * j * SEG
        sb = sa + SEG
        pltpu.make_async_copy(
            src_hbm.at[pl.ds(sa, SEG)], sidx_a, sem_a).wait()
        pltpu.make_async_copy(
            dst_hbm.at[pl.ds(sa, SEG)], didx_a, sem_a).wait()
        pltpu.async_copy(src_hbm.at[pl.ds(sb, SEG)], sidx_b, sem_b)
        pltpu.async_copy(dst_hbm.at[pl.ds(sb, SEG)], didx_b, sem_b)
        process(sidx_a, didx_a)

        @pl.when(j + 1 < npair)
        def _():
            sn = sa + 2 * SEG
            pltpu.async_copy(src_hbm.at[pl.ds(sn, SEG)], sidx_a, sem_a)
            pltpu.async_copy(dst_hbm.at[pl.ds(sn, SEG)], didx_a, sem_a)

        pltpu.make_async_copy(
            src_hbm.at[pl.ds(sb, SEG)], sidx_b, sem_b).wait()
        pltpu.make_async_copy(
            dst_hbm.at[pl.ds(sb, SEG)], didx_b, sem_b).wait()
        process(sidx_b, didx_b)
        return carry

    lax.fori_loop(0, npair, body, 0)
    plsc.subcore_barrier()
    for off in (0, K, 2 * K, 3 * K, 4 * K):
        c = min(K, STRIPE - off)
        pltpu.sync_copy(acc.at[pl.ds(sid * STRIPE + off, c)],
                        rows0.at[pl.ds(0, c)])
        pltpu.sync_copy(rows0.at[pl.ds(0, c)],
                        out_hbm.at[cid, pl.ds(sid * STRIPE + off, c)])


def kernel(x, edge_index, batch, W_pre1, b_pre1, W_pre2, b_pre2,
           W_g0, b_g0, W_g1, b_g1, W_g2, b_g2, W_post):
    src = edge_index[0].astype(jnp.int32)
    dst = edge_index[1].astype(jnp.int32)
    padn = EP - E
    pad_src = (jnp.arange(padn, dtype=jnp.int32) * 37) % N
    pad_dst = N + jnp.arange(padn, dtype=jnp.int32) % (NP - N)
    src2 = jnp.concatenate([src, pad_src]).reshape(EC, K)
    dst2 = jnp.concatenate([dst, pad_dst]).reshape(EC, K)
    zeros_stripe = jnp.zeros((K, 128), jnp.float32)
    deg_init = jnp.concatenate([jnp.ones((STRIPE,), jnp.float32),
                                jnp.zeros((STRIPE,), jnp.float32)])
    batch_p = jnp.concatenate([
        batch.astype(jnp.int32),
        jnp.full((GRID * BLK - N,), 127, jnp.int32)]).reshape(GRID * BLK, 1)
    b1 = b_pre1.reshape(1, -1)
    b2 = b_pre2.reshape(1, -1)
    bg0 = b_g0.reshape(1, -1)
    bg1 = b_g1.reshape(1, -1)
    bg2 = b_g2.reshape(1, -1)

    degp = _deg_call(dst2, deg_init)
    deg_t = jnp.transpose(degp.reshape(NSC, NP))
    h0, p0 = _pre_call(x, W_pre1, b1, W_pre2, b2, deg_t, W_g0)
    zp0 = _msg_call(p0, src2, dst2, zeros_stripe)
    h1, p1 = _mk_call(zp0, p0, deg_t, h0, bg0, W_g1)
    zp1 = _msg_call(p1, src2, dst2, zeros_stripe)
    h2, p2 = _mk_call(zp1, p1, deg_t, h1, bg1, W_g2)
    zp2 = _msg_call(p2, src2, dst2, zeros_stripe)
    pred = _final_call(zp2, p2, deg_t, h2, bg2, batch_p, W_post)
    return pred.reshape(-1, 10)
